# Initial kernel scaffold; baseline (speedup 1.0000x reference)
import jax
import jax.numpy as jnp
from jax.experimental import pallas as pl


def kernel(observations, piece_ids, legal_actions, W1, b1, W2, b2, piece_emb):
    raise NotImplementedError("write your pallas kernel here")


# speedup vs baseline: 1.7918x; 1.7918x over previous
import functools

import jax
import jax.numpy as jnp
from jax.experimental import pallas as pl
from jax.experimental.pallas import tpu as pltpu

OBS_DIM = 4096
HIDDEN = 2048
N_ACTIONS = 4096
N_PIECES = 32
PIECE_VOCAB = 64
BATCH = 128
THRESHOLD = 0.001

H_BLK = 512
A_BLK = 512


def _h_kernel(obs_ref, pid_ref, w1_ref, b1_ref, pe_ref, h_ref):
    ids = pid_ref[...]
    iota = jax.lax.broadcasted_iota(jnp.int32, (BATCH, N_PIECES, PIECE_VOCAB), 2)
    counts = jnp.sum((ids[:, :, None] == iota).astype(jnp.float32), axis=1)
    pe = jnp.dot(counts, pe_ref[...], preferred_element_type=jnp.float32,
                 precision=jax.lax.Precision.HIGHEST)
    acc = jnp.dot(obs_ref[...].astype(jnp.bfloat16),
                  w1_ref[...].astype(jnp.bfloat16),
                  preferred_element_type=jnp.float32)
    h_ref[...] = jnp.maximum(acc + b1_ref[...] + pe, 0.0)


def _logits_sample_kernel(h_ref, w2_ref, b2_ref, legal_ref, g_ref,
                          lp_ref, act_ref):
    i = pl.program_id(0)
    n = pl.num_programs(0)
    blk = jnp.dot(h_ref[...].astype(jnp.bfloat16),
                  w2_ref[...].astype(jnp.bfloat16),
                  preferred_element_type=jnp.float32)
    blk = blk + b2_ref[...]
    blk = jnp.where(legal_ref[...] > 0, blk, jnp.float32(-1e9))
    lp_ref[:, pl.ds(i * A_BLK, A_BLK)] = blk

    @pl.when(i == n - 1)
    def _finalize():
        masked = lp_ref[...]
        m = jnp.max(masked, axis=1, keepdims=True)
        shifted = masked - m
        lse = jnp.log(jnp.sum(jnp.exp(shifted), axis=1, keepdims=True))
        log_probs = shifted - lse
        lp_ref[...] = log_probs
        probs = jnp.exp(log_probs)
        probs = jnp.where(probs > THRESHOLD, probs, 0.0)
        probs = probs / jnp.sum(probs, axis=1, keepdims=True)
        scores = jnp.log(jnp.clip(probs, 1e-30, None)) + g_ref[...]
        smax = jnp.max(scores, axis=1, keepdims=True)
        idx = jax.lax.broadcasted_iota(jnp.int32, (BATCH, N_ACTIONS), 1)
        cand = jnp.where(scores == smax, idx, N_ACTIONS)
        act_ref[0, :] = jnp.min(cand, axis=1)


@functools.partial(jax.jit, static_argnames=("interpret",))
def kernel(observations, piece_ids, legal_actions, W1, b1, W2, b2, piece_emb,
           interpret=False):
    piece_ids = piece_ids.astype(jnp.int32)
    b1_2d = b1.reshape(1, HIDDEN)
    b2_2d = b2.reshape(1, N_ACTIONS)
    gumbel = jax.random.gumbel(jax.random.key(42), (BATCH, N_ACTIONS),
                               jnp.float32)

    h = pl.pallas_call(
        _h_kernel,
        grid=(HIDDEN // H_BLK,),
        in_specs=[
            pl.BlockSpec((BATCH, OBS_DIM), lambda j: (0, 0)),
            pl.BlockSpec((BATCH, N_PIECES), lambda j: (0, 0)),
            pl.BlockSpec((OBS_DIM, H_BLK), lambda j: (0, j)),
            pl.BlockSpec((1, H_BLK), lambda j: (0, j)),
            pl.BlockSpec((PIECE_VOCAB, H_BLK), lambda j: (0, j)),
        ],
        out_specs=pl.BlockSpec((BATCH, H_BLK), lambda j: (0, j)),
        out_shape=jax.ShapeDtypeStruct((BATCH, HIDDEN), jnp.float32),
        interpret=interpret,
    )(observations, piece_ids, W1, b1_2d, piece_emb)

    log_probs, action = pl.pallas_call(
        _logits_sample_kernel,
        grid=(N_ACTIONS // A_BLK,),
        in_specs=[
            pl.BlockSpec((BATCH, HIDDEN), lambda i: (0, 0)),
            pl.BlockSpec((HIDDEN, A_BLK), lambda i: (0, i)),
            pl.BlockSpec((1, A_BLK), lambda i: (0, i)),
            pl.BlockSpec((BATCH, A_BLK), lambda i: (0, i)),
            pl.BlockSpec((BATCH, N_ACTIONS), lambda i: (0, 0)),
        ],
        out_specs=[
            pl.BlockSpec((BATCH, N_ACTIONS), lambda i: (0, 0)),
            pl.BlockSpec((1, BATCH), lambda i: (0, 0)),
        ],
        out_shape=[
            jax.ShapeDtypeStruct((BATCH, N_ACTIONS), jnp.float32),
            jax.ShapeDtypeStruct((1, BATCH), jnp.int32),
        ],
        interpret=interpret,
    )(h, W2, b2_2d, legal_actions, gumbel)

    return (log_probs, action.reshape(BATCH))
